# Initial kernel scaffold; baseline (speedup 1.0000x reference)
import jax
import jax.numpy as jnp
from jax.experimental import pallas as pl


def kernel(x, breakpoints, intercepts, signs, exps):
    raise NotImplementedError("write your pallas kernel here")


# speedup vs baseline: 2.6405x; 2.6405x over previous
import jax
import jax.numpy as jnp
from jax.experimental import pallas as pl
from jax.experimental.pallas import tpu as pltpu

_SCALE = 65536.0
_NSEG = 16
_ROWS = 2048
_COLS = 8192
_BR = 128


def _body(bp_ref, d_ref, pk0_ref, x_ref, o_ref):
    xq = (x_ref[...] * _SCALE).astype(jnp.int32)
    p = jnp.where(xq >= bp_ref[0], d_ref[0], 0) + pk0_ref[0]
    for i in range(1, _NSEG - 1):
        p = p + jnp.where(xq >= bp_ref[i], d_ref[i], 0)
    e = p & 7
    neg = -((p >> 3) & 1)
    inter = (p >> 4) - 65536
    sh = jnp.right_shift(xq, e)
    y = inter + ((sh ^ neg) - neg)
    o_ref[...] = y.astype(jnp.float32) * (1.0 / _SCALE)


def kernel(x, breakpoints, intercepts, signs, exps):
    sneg = (signs < 0).astype(jnp.int32)
    packed = ((intercepts + 65536) << 4) | (sneg << 3) | exps
    pk0 = packed[:1]
    d = packed[1:] - packed[:-1]

    x2 = x.reshape(_ROWS, _COLS)
    out = pl.pallas_call(
        _body,
        grid=(_ROWS // _BR,),
        in_specs=[
            pl.BlockSpec(memory_space=pltpu.SMEM),
            pl.BlockSpec(memory_space=pltpu.SMEM),
            pl.BlockSpec(memory_space=pltpu.SMEM),
            pl.BlockSpec((_BR, _COLS), lambda i: (i, 0)),
        ],
        out_specs=pl.BlockSpec((_BR, _COLS), lambda i: (i, 0)),
        out_shape=jax.ShapeDtypeStruct((_ROWS, _COLS), jnp.float32),
    )(breakpoints, d, pk0, x2)
    return out.reshape(x.shape)


# speedup vs baseline: 3.0572x; 1.1578x over previous
import jax
import jax.numpy as jnp
from jax.experimental import pallas as pl
from jax.experimental.pallas import tpu as pltpu

_SCALE = 65536.0
_ROWS = 2048
_COLS = 8192
_BR = 128


def _body(t_ref, pk_ref, x_ref, o_ref):
    v = x_ref[...] * _SCALE
    xq = v.astype(jnp.int32)
    T = [t_ref[i] for i in range(15)]
    P = [pk_ref[i] for i in range(16)]
    sel = jnp.where
    c1 = v >= T[7]
    c2 = v >= sel(c1, T[11], T[3])
    c3 = v >= sel(c2, sel(c1, T[13], T[5]), sel(c1, T[9], T[1]))
    c4 = v >= sel(c3,
                  sel(c2, sel(c1, T[14], T[6]), sel(c1, T[10], T[2])),
                  sel(c2, sel(c1, T[12], T[4]), sel(c1, T[8], T[0])))
    q = [sel(c4, P[2 * k + 1], P[2 * k]) for k in range(8)]
    r = [sel(c3, q[2 * k + 1], q[2 * k]) for k in range(4)]
    s = [sel(c2, r[2 * k + 1], r[2 * k]) for k in range(2)]
    p = sel(c1, s[1], s[0])
    e = p & 7
    negm = (p << 28) >> 31
    inter = p >> 4
    sh = jnp.right_shift(xq, e)
    y = inter + ((sh ^ negm) - negm)
    o_ref[...] = y.astype(jnp.float32) * (1.0 / _SCALE)


def kernel(x, breakpoints, intercepts, signs, exps):
    sneg = (signs < 0).astype(jnp.int32)
    packed = (intercepts << 4) | (sneg << 3) | exps
    t = jnp.where(breakpoints >= 1,
                  breakpoints.astype(jnp.float32),
                  jnp.float32(-0.99999994))

    x2 = x.reshape(_ROWS, _COLS)
    out = pl.pallas_call(
        _body,
        grid=(_ROWS // _BR,),
        in_specs=[
            pl.BlockSpec(memory_space=pltpu.SMEM),
            pl.BlockSpec(memory_space=pltpu.SMEM),
            pl.BlockSpec((_BR, _COLS), lambda i: (i, 0)),
        ],
        out_specs=pl.BlockSpec((_BR, _COLS), lambda i: (i, 0)),
        out_shape=jax.ShapeDtypeStruct((_ROWS, _COLS), jnp.float32),
    )(t, packed, x2)
    return out.reshape(x.shape)


# speedup vs baseline: 7.1090x; 2.3253x over previous
import jax
import jax.numpy as jnp
from jax.experimental import pallas as pl
from jax.experimental.pallas import tpu as pltpu

_SCALE = 65536.0
_N = 16777216
_BLK = 1048576


def _body(t_ref, pk_ref, x_ref, o_ref):
    v = x_ref[...] * _SCALE
    xq = v.astype(jnp.int32)
    T = [t_ref[i] for i in range(15)]
    P = [pk_ref[i] for i in range(16)]
    sel = jnp.where
    c1 = v >= T[7]
    c2 = v >= sel(c1, T[11], T[3])
    c3 = v >= sel(c2, sel(c1, T[13], T[5]), sel(c1, T[9], T[1]))
    c4 = v >= sel(c3,
                  sel(c2, sel(c1, T[14], T[6]), sel(c1, T[10], T[2])),
                  sel(c2, sel(c1, T[12], T[4]), sel(c1, T[8], T[0])))
    q = [sel(c4, P[2 * k + 1], P[2 * k]) for k in range(8)]
    r = [sel(c3, q[2 * k + 1], q[2 * k]) for k in range(4)]
    s = [sel(c2, r[2 * k + 1], r[2 * k]) for k in range(2)]
    p = sel(c1, s[1], s[0])
    e = p & 7
    negm = (p << 28) >> 31
    inter = p >> 4
    sh = jnp.right_shift(xq, e)
    y = inter + ((sh ^ negm) - negm)
    o_ref[...] = y.astype(jnp.float32) * (1.0 / _SCALE)


def kernel(x, breakpoints, intercepts, signs, exps):
    sneg = (signs < 0).astype(jnp.int32)
    packed = (intercepts << 4) | (sneg << 3) | exps
    t = jnp.where(breakpoints >= 1,
                  breakpoints.astype(jnp.float32),
                  jnp.float32(-0.99999994))

    out = pl.pallas_call(
        _body,
        grid=(_N // _BLK,),
        in_specs=[
            pl.BlockSpec(memory_space=pltpu.SMEM),
            pl.BlockSpec(memory_space=pltpu.SMEM),
            pl.BlockSpec((_BLK,), lambda i: (i,)),
        ],
        out_specs=pl.BlockSpec((_BLK,), lambda i: (i,)),
        out_shape=jax.ShapeDtypeStruct((_N,), jnp.float32),
    )(t, packed, x)
    return out
